# Initial kernel scaffold; baseline (speedup 1.0000x reference)
import jax
import jax.numpy as jnp
from jax.experimental import pallas as pl


def kernel(params, param_table, logits):
    raise NotImplementedError("write your pallas kernel here")


# speedup vs baseline: 2.1477x; 2.1477x over previous
import math

import jax
import jax.numpy as jnp
from jax.experimental import pallas as pl
from jax.experimental.pallas import tpu as pltpu

_BLK = 512


def _vq_body(p_ref, tab_ref, logit_ref, lpmf_ref, pq_ref, bit_ref):
    i = pl.program_id(0)
    p = jnp.clip(p_ref[...], -1.0, 1.0)
    tab = tab_ref[...]
    nlevel = tab.shape[0]

    scores = jax.lax.dot_general(
        p, tab, (((1,), (1,)), ((), ())),
        preferred_element_type=jnp.float32)
    tnorm = jnp.sum(tab * tab, axis=1)
    neg = 2.0 * scores - tnorm[None, :]

    m = jnp.max(neg, axis=1, keepdims=True)
    iota = jax.lax.broadcasted_iota(jnp.int32, neg.shape, 1)
    idx = jnp.min(jnp.where(neg >= m, iota, nlevel), axis=1)
    oh = (iota == idx[:, None]).astype(jnp.float32)

    q = jax.lax.dot_general(
        oh, tab, (((1,), (0,)), ((), ())),
        preferred_element_type=jnp.float32)
    mq = jnp.max(q, axis=1, keepdims=True)
    lse = jnp.log(jnp.sum(jnp.exp(q - mq), axis=1, keepdims=True)) + mq
    lpmf_ref[...] = q - lse
    pq_ref[...] = q

    lg = logit_ref[...]
    ml = jnp.max(lg)
    llc = lg - (jnp.log(jnp.sum(jnp.exp(lg - ml))) + ml)
    bit_blk = jnp.sum(oh * llc) * (-1.0 / math.log(2.0))

    @pl.when(i == 0)
    def _():
        bit_ref[0, 0] = bit_blk

    @pl.when(i > 0)
    def _():
        bit_ref[0, 0] += bit_blk


def kernel(params, param_table, logits):
    a, b, c = params.shape
    d = param_table.shape[0]
    tokens = a * b
    p2 = params.reshape(tokens, c)
    lg2 = logits.reshape(1, d)
    grid = tokens // _BLK

    lpmf, pq, bit = pl.pallas_call(
        _vq_body,
        grid=(grid,),
        in_specs=[
            pl.BlockSpec((_BLK, c), lambda i: (i, 0)),
            pl.BlockSpec((d, c), lambda i: (0, 0)),
            pl.BlockSpec((1, d), lambda i: (0, 0)),
        ],
        out_specs=[
            pl.BlockSpec((_BLK, c), lambda i: (i, 0)),
            pl.BlockSpec((_BLK, c), lambda i: (i, 0)),
            pl.BlockSpec(memory_space=pltpu.SMEM),
        ],
        out_shape=[
            jax.ShapeDtypeStruct((tokens, c), jnp.float32),
            jax.ShapeDtypeStruct((tokens, c), jnp.float32),
            jax.ShapeDtypeStruct((1, 1), jnp.float32),
        ],
        compiler_params=pltpu.CompilerParams(
            dimension_semantics=("arbitrary",),
        ),
    )(p2, param_table, lg2)

    return (lpmf.reshape(a, b, c), pq.reshape(a, b, c), bit[0, 0])
